# baseline (speedup 1.0000x reference)
import functools

import jax
import jax.numpy as jnp
from jax import lax
from jax.experimental import pallas as pl
from jax.experimental.pallas import tpu as pltpu
from jax.experimental.pallas import tpu_sc as plsc

BATCH = 16384
EMB_K = 64
NW = 32
B_PER_W = BATCH // NW
IDX_CHUNK = 128
N_CHUNKS = B_PER_W // IDX_CHUNK
GROUPS = B_PER_W // 16


def _sc_kernel(xu_hbm, xi_hbm, wb_hbm, user_hbm, item_hbm, out_hbm,
               idx_u, idx_v, urows, irows, wb_v, out_v, sem):
    cid = lax.axis_index("c")
    sid = lax.axis_index("s")
    wid = sid * 2 + cid
    pltpu.sync_copy(xu_hbm.at[pl.ds(wid * N_CHUNKS, N_CHUNKS)], idx_u)
    pltpu.sync_copy(xi_hbm.at[pl.ds(wid * N_CHUNKS, N_CHUNKS)], idx_v)
    pltpu.sync_copy(wb_hbm, wb_v)
    copies = []
    for j in range(N_CHUNKS):
        copies.append(pltpu.async_copy(
            user_hbm.at[idx_u.at[j]],
            urows.at[pl.ds(j * IDX_CHUNK, IDX_CHUNK)], sem))
        copies.append(pltpu.async_copy(
            item_hbm.at[idx_v.at[j]],
            irows.at[pl.ds(j * IDX_CHUNK, IDX_CHUNK)], sem))
    for c in copies:
        c.wait()

    iota = lax.iota(jnp.int32, 16)

    def group_body(g, _):
        row0 = g * 16
        ridx = row0 + iota
        acc = wb_v[2 * EMB_K]
        for k in range(EMB_K):
            col = jnp.full((16,), k, jnp.int32)
            uval = plsc.load_gather(urows, [ridx, col])
            acc = acc + uval * wb_v[k]
            ival = plsc.load_gather(irows, [ridx, col])
            acc = acc + ival * wb_v[EMB_K + k]
        out_v[pl.ds(row0, 16)] = 1.0 / (1.0 + jnp.exp(-acc))
        return 0

    lax.fori_loop(0, GROUPS, group_body, 0)
    pltpu.sync_copy(out_v, out_hbm.at[pl.ds(wid * B_PER_W, B_PER_W)])


@jax.jit
def _run(xu, xi, wb, user_table, item_table):
    mesh = plsc.VectorSubcoreMesh(core_axis_name="c", subcore_axis_name="s")
    kfn = functools.partial(
        pl.kernel,
        mesh=mesh,
        compiler_params=pltpu.CompilerParams(
            needs_layout_passes=False, use_tc_tiling_on_sc=False),
        out_type=jax.ShapeDtypeStruct((BATCH,), jnp.float32),
        scratch_types=[
            pltpu.VMEM((N_CHUNKS, IDX_CHUNK), jnp.int32),
            pltpu.VMEM((N_CHUNKS, IDX_CHUNK), jnp.int32),
            pltpu.VMEM((B_PER_W, EMB_K), jnp.float32),
            pltpu.VMEM((B_PER_W, EMB_K), jnp.float32),
            pltpu.VMEM((2 * EMB_K + 8, 16), jnp.float32),
            pltpu.VMEM((B_PER_W,), jnp.float32),
            pltpu.SemaphoreType.DMA,
        ],
    )(_sc_kernel)
    return kfn(xu, xi, wb, user_table, item_table)


def kernel(x, user_table, item_table, W, b):
    xu = x[:, 0].astype(jnp.int32).reshape(NW * N_CHUNKS, IDX_CHUNK)
    xi = x[:, 1].astype(jnp.int32).reshape(NW * N_CHUNKS, IDX_CHUNK)
    wflat = W.reshape(2 * EMB_K)
    wb = jnp.concatenate([wflat, b, jnp.zeros((7,), jnp.float32)])
    wb = jnp.broadcast_to(wb[:, None], (2 * EMB_K + 8, 16))
    return _run(xu, xi, wb, user_table, item_table)


# speedup vs baseline: 6.3866x; 6.3866x over previous
import functools

import jax
import jax.numpy as jnp
from jax import lax
from jax.experimental import pallas as pl
from jax.experimental.pallas import tpu as pltpu
from jax.experimental.pallas import tpu_sc as plsc

BATCH = 16384
EMB_K = 64
NROWS = 1000000
NW = 32
B_PER_W = BATCH // NW
IDX_CHUNK = 128
N_CHUNKS = B_PER_W // IDX_CHUNK
BL = 16384
GRID = -(-NROWS // BL)


def _matvec_kernel(ut_ref, it_ref, wu_ref, wi_ref, pu_ref, pi_ref):
    pu_ref[...] = jnp.sum(ut_ref[...] * wu_ref[...], axis=0)
    pi_ref[...] = jnp.sum(it_ref[...] * wi_ref[...], axis=0)


def _project(ut, it, wu_col, wi_col):
    return pl.pallas_call(
        _matvec_kernel,
        grid=(GRID,),
        in_specs=[
            pl.BlockSpec((EMB_K, BL), lambda i: (0, i)),
            pl.BlockSpec((EMB_K, BL), lambda i: (0, i)),
            pl.BlockSpec((EMB_K, 1), lambda i: (0, 0)),
            pl.BlockSpec((EMB_K, 1), lambda i: (0, 0)),
        ],
        out_specs=[
            pl.BlockSpec((BL,), lambda i: (i,)),
            pl.BlockSpec((BL,), lambda i: (i,)),
        ],
        out_shape=[
            jax.ShapeDtypeStruct((NROWS,), jnp.float32),
            jax.ShapeDtypeStruct((NROWS,), jnp.float32),
        ],
    )(ut, it, wu_col, wi_col)


def _sc_kernel(xu_hbm, xi_hbm, b_hbm, pu_hbm, pi_hbm, out_hbm,
               idx_u, idx_i, pu_v, pi_v, b_v, out_v, *sems):
    cid = lax.axis_index("c")
    sid = lax.axis_index("s")
    wid = sid * 2 + cid
    pltpu.sync_copy(xu_hbm.at[pl.ds(wid * N_CHUNKS, N_CHUNKS)], idx_u)
    pltpu.sync_copy(xi_hbm.at[pl.ds(wid * N_CHUNKS, N_CHUNKS)], idx_i)
    pltpu.sync_copy(b_hbm, b_v)
    copies = []
    for j in range(N_CHUNKS):
        copies.append(pltpu.async_copy(
            pu_hbm.at[idx_u.at[j]],
            pu_v.at[pl.ds(j * IDX_CHUNK, IDX_CHUNK)], sems[j]))
        copies.append(pltpu.async_copy(
            pi_hbm.at[idx_i.at[j]],
            pi_v.at[pl.ds(j * IDX_CHUNK, IDX_CHUNK)], sems[N_CHUNKS + j]))
    for c in copies:
        c.wait()
    bias = b_v[...]
    for g in range(B_PER_W // 16):
        z = pu_v[pl.ds(g * 16, 16)] + pi_v[pl.ds(g * 16, 16)] + bias
        out_v[pl.ds(g * 16, 16)] = 1.0 / (1.0 + jnp.exp(-z))
    pltpu.sync_copy(out_v, out_hbm.at[pl.ds(wid * B_PER_W, B_PER_W)])


def _gather_head(xu, xi, b16, pu, pi):
    mesh = plsc.VectorSubcoreMesh(core_axis_name="c", subcore_axis_name="s")
    kfn = functools.partial(
        pl.kernel,
        mesh=mesh,
        compiler_params=pltpu.CompilerParams(
            needs_layout_passes=False, use_tc_tiling_on_sc=False),
        out_type=jax.ShapeDtypeStruct((BATCH,), jnp.float32),
        scratch_types=[
            pltpu.VMEM((N_CHUNKS, IDX_CHUNK), jnp.int32),
            pltpu.VMEM((N_CHUNKS, IDX_CHUNK), jnp.int32),
            pltpu.VMEM((B_PER_W,), jnp.float32),
            pltpu.VMEM((B_PER_W,), jnp.float32),
            pltpu.VMEM((16,), jnp.float32),
            pltpu.VMEM((B_PER_W,), jnp.float32),
        ] + [pltpu.SemaphoreType.DMA] * (2 * N_CHUNKS),
    )(_sc_kernel)
    return kfn(xu, xi, b16, pu, pi)


@jax.jit
def _run(x, user_table, item_table, W, b):
    xu = x[:, 0].astype(jnp.int32).reshape(NW * N_CHUNKS, IDX_CHUNK)
    xi = x[:, 1].astype(jnp.int32).reshape(NW * N_CHUNKS, IDX_CHUNK)
    wu_col = W[0, :EMB_K].reshape(EMB_K, 1)
    wi_col = W[0, EMB_K:].reshape(EMB_K, 1)
    b16 = jnp.broadcast_to(b, (16,))
    pu, pi = _project(user_table.T, item_table.T, wu_col, wi_col)
    return _gather_head(xu, xi, b16, pu, pi)


def kernel(x, user_table, item_table, W, b):
    return _run(x, user_table, item_table, W, b)
